# baseline (speedup 1.0000x reference)
import jax
import jax.numpy as jnp
from jax import lax
from jax.experimental import pallas as pl
from jax.experimental.pallas import tpu as pltpu
from jax.experimental.pallas import tpu_sc as plsc

VOCAB = 1000000
LENGTH = 200
EMBED = 32
BATCH = 4096

NW = 32
CHUNK = 128
TOKENS = BATCH * LENGTH
PER_W = TOKENS // NW
NCHUNK = PER_W // CHUNK
LANES = 16
NBT = BATCH // CHUNK
DT = EMBED // 8
NBUF = 4


def _emb_body(ids_hbm, pos_hbm, table_hbm, out_hbm,
              idx_v, pos_v, rows_v, out_v, gsems, osems):
    wid = lax.axis_index("s") * 2 + lax.axis_index("c")
    pltpu.sync_copy(ids_hbm.at[pl.ds(wid * NCHUNK, NCHUNK)], idx_v)
    pltpu.sync_copy(pos_hbm, pos_v)

    iota16 = lax.iota(jnp.int32, 16)
    fidx = [[lax.shift_right_logical(iota16 + h * LANES, 3) * 1024
             + lax.bitwise_and(iota16 + h * LANES, 7) * CHUNK + u
             for u in range(8)] for h in range(2)]

    def chunk_lbt(cc):
        g = wid * NCHUNK + cc
        l = lax.div(g, 8 * NBT) * 8 + lax.rem(g, 8)
        bt = lax.rem(lax.div(g, 8), NBT)
        return l, bt

    def gather(cc, p):
        return pltpu.make_async_copy(
            table_hbm.at[idx_v.at[cc]], rows_v[p], gsems[p])

    def wb(cc, p):
        l, bt = chunk_lbt(cc)
        return [pltpu.make_async_copy(
                    out_v[p].at[pl.ds(dt * 1024, 1024)],
                    out_hbm.at[l, pl.ds((dt * NBT + bt) * 1024, 1024)],
                    osems[p])
                for dt in range(DT)]

    for k in range(NBUF - 1):
        gather(k, k).start()

    def step(i, carry):
        for b in range(NBUF):
            cc = i * NBUF + b

            @pl.when(cc >= NBUF)
            def _drain():
                for c in wb(cc - NBUF, b):
                    c.wait()

            @pl.when(cc + NBUF - 1 < NCHUNK)
            def _prefetch():
                gather(cc + NBUF - 1, (b + NBUF - 1) % NBUF).start()

            gather(cc, b).wait()

            l, _ = chunk_lbt(cc)
            pos_c = [pos_v[pl.ds(l * EMBED + h * LANES, LANES)]
                     for h in range(2)]

            def tok_body(jj, c2):
                bjj = jnp.full((LANES,), jj * 8, jnp.int32)
                for u in range(8):
                    for h in range(2):
                        val = rows_v[b][jj * 8 + u,
                                        pl.ds(h * LANES, LANES)] + pos_c[h]
                        plsc.store_scatter(out_v[b], [fidx[h][u] + bjj], val)
                return c2

            lax.fori_loop(0, CHUNK // 8, tok_body, 0)

            for c in wb(cc, b):
                c.start()
        return carry

    lax.fori_loop(0, NCHUNK // NBUF, step, 0)

    for k in range(NBUF):
        cc = NCHUNK - NBUF + k
        for c in wb(cc, cc % NBUF):
            c.wait()


@jax.jit
def _emb_call(ids, pos, table4):
    mesh = plsc.VectorSubcoreMesh(core_axis_name="c", subcore_axis_name="s")
    f = pl.kernel(
        _emb_body,
        out_type=jax.ShapeDtypeStruct((LENGTH, DT * NBT * 8 * CHUNK),
                                      jnp.float32),
        mesh=mesh,
        compiler_params=pltpu.CompilerParams(use_tc_tiling_on_sc=False,
                                             needs_layout_passes=False),
        scratch_types=[
            pltpu.VMEM((NCHUNK, CHUNK), jnp.int32),
            pltpu.VMEM((LENGTH * EMBED,), jnp.float32),
            [pltpu.VMEM((CHUNK, EMBED), jnp.float32) for _ in range(NBUF)],
            [pltpu.VMEM((DT * 8 * CHUNK,), jnp.float32) for _ in range(NBUF)],
            [pltpu.SemaphoreType.DMA for _ in range(NBUF)],
            [pltpu.SemaphoreType.DMA for _ in range(NBUF)],
        ],
    )
    return f(ids, pos, table4)


def kernel(input_ids, token_table, position_table):
    ids = ((input_ids.astype(jnp.int32) * 4).T
           .reshape(LENGTH // 8, 8, NBT, CHUNK)
           .transpose(0, 2, 1, 3)
           .reshape(TOKENS // CHUNK, CHUNK))
    pos = position_table.reshape(LENGTH * EMBED)
    table4 = jnp.pad(token_table, ((0, 0), (0, 96))).reshape(4 * VOCAB, EMBED)
    out2 = _emb_call(ids, pos, table4)
    out5 = out2.reshape(LENGTH, DT, NBT, 8, CHUNK)
    return out5.transpose(2, 4, 0, 1, 3).reshape(BATCH, LENGTH, EMBED)


# speedup vs baseline: 1.1286x; 1.1286x over previous
import jax
import jax.numpy as jnp
from jax import lax
from jax.experimental import pallas as pl
from jax.experimental.pallas import tpu as pltpu
from jax.experimental.pallas import tpu_sc as plsc

VOCAB = 1000000
LENGTH = 200
EMBED = 32
BATCH = 4096

NW = 32
CHUNK = 128
TOKENS = BATCH * LENGTH
PER_W = TOKENS // NW
NCHUNK = PER_W // CHUNK
LANES = 16
NBT = BATCH // CHUNK
DT = EMBED // 8
NBUF = 4


def _emb_body(ids_hbm, pos_hbm, table_hbm, out_hbm,
              idx_v, pos_v, rows_v, out_v, gsems, osems):
    wid = lax.axis_index("s") * 2 + lax.axis_index("c")
    pltpu.sync_copy(ids_hbm.at[pl.ds(wid * NCHUNK, NCHUNK)], idx_v)
    pltpu.sync_copy(pos_hbm, pos_v)

    iota16 = lax.iota(jnp.int32, 16)
    fidx = [[lax.shift_right_logical(iota16 + h * LANES, 3) * 1024
             + lax.bitwise_and(iota16 + h * LANES, 7) * CHUNK + u
             for u in range(8)] for h in range(2)]

    def chunk_lbt(cc):
        g = wid * NCHUNK + cc
        l = lax.div(g, 8 * NBT) * 8 + lax.rem(g, 8)
        bt = lax.rem(lax.div(g, 8), NBT)
        return l, bt

    def gather(cc, p):
        return pltpu.make_async_copy(
            table_hbm.at[idx_v.at[cc]], rows_v[p], gsems[p])

    def wb(cc, p):
        l, bt = chunk_lbt(cc)
        return [pltpu.make_async_copy(
                    out_v[p].at[pl.ds(dt * 1024, 1024)],
                    out_hbm.at[l, pl.ds((dt * NBT + bt) * 1024, 1024)],
                    osems[p])
                for dt in range(DT)]

    for k in range(NBUF - 1):
        gather(k, k).start()

    def step(i, carry):
        for b in range(NBUF):
            cc = i * NBUF + b

            @pl.when(cc >= NBUF)
            def _drain():
                for c in wb(cc - NBUF, b):
                    c.wait()

            @pl.when(cc + NBUF - 1 < NCHUNK)
            def _prefetch():
                gather(cc + NBUF - 1, (b + NBUF - 1) % NBUF).start()

            gather(cc, b).wait()

            l, _ = chunk_lbt(cc)
            pos_c = [pos_v[pl.ds(l * EMBED + h * LANES, LANES)]
                     for h in range(2)]

            def tok_body(jj, c2):
                bjj = jnp.full((LANES,), jj * 8, jnp.int32)
                vals = [rows_v[b][jj * 8 + u, pl.ds(h * LANES, LANES)]
                        + pos_c[h]
                        for u in range(8) for h in range(2)]
                for u in range(8):
                    for h in range(2):
                        plsc.store_scatter(out_v[b], [fidx[h][u] + bjj],
                                           vals[u * 2 + h])
                return c2

            lax.fori_loop(0, CHUNK // 8, tok_body, 0)

            for c in wb(cc, b):
                c.start()
        return carry

    lax.fori_loop(0, NCHUNK // NBUF, step, 0)

    for k in range(NBUF):
        cc = NCHUNK - NBUF + k
        for c in wb(cc, cc % NBUF):
            c.wait()


@jax.jit
def _emb_call(ids, pos, table4):
    mesh = plsc.VectorSubcoreMesh(core_axis_name="c", subcore_axis_name="s")
    f = pl.kernel(
        _emb_body,
        out_type=jax.ShapeDtypeStruct((LENGTH, DT * NBT * 8 * CHUNK),
                                      jnp.float32),
        mesh=mesh,
        compiler_params=pltpu.CompilerParams(use_tc_tiling_on_sc=False,
                                             needs_layout_passes=False),
        scratch_types=[
            pltpu.VMEM((NCHUNK, CHUNK), jnp.int32),
            pltpu.VMEM((LENGTH * EMBED,), jnp.float32),
            [pltpu.VMEM((CHUNK, EMBED), jnp.float32) for _ in range(NBUF)],
            [pltpu.VMEM((DT * 8 * CHUNK,), jnp.float32) for _ in range(NBUF)],
            [pltpu.SemaphoreType.DMA for _ in range(NBUF)],
            [pltpu.SemaphoreType.DMA for _ in range(NBUF)],
        ],
    )
    return f(ids, pos, table4)


def kernel(input_ids, token_table, position_table):
    ids = ((input_ids.astype(jnp.int32) * 4).T
           .reshape(LENGTH // 8, 8, NBT, CHUNK)
           .transpose(0, 2, 1, 3)
           .reshape(TOKENS // CHUNK, CHUNK))
    pos = position_table.reshape(LENGTH * EMBED)
    table4 = jnp.pad(token_table, ((0, 0), (0, 96))).reshape(4 * VOCAB, EMBED)
    out2 = _emb_call(ids, pos, table4)
    out5 = out2.reshape(LENGTH, DT, NBT, 8, CHUNK)
    return out5.transpose(2, 4, 0, 1, 3).reshape(BATCH, LENGTH, EMBED)


# speedup vs baseline: 1.1619x; 1.0295x over previous
import jax
import jax.numpy as jnp
from jax import lax
from jax.experimental import pallas as pl
from jax.experimental.pallas import tpu as pltpu
from jax.experimental.pallas import tpu_sc as plsc

VOCAB = 1000000
LENGTH = 200
EMBED = 32
BATCH = 4096

NW = 32
CHUNK = 128
TOKENS = BATCH * LENGTH
PER_W = TOKENS // NW
NCHUNK = PER_W // CHUNK
LANES = 16
NBT = BATCH // CHUNK
DT = EMBED // 8
NBUF = 4


def _emb_body(ids_hbm, pos_hbm, table_hbm, out_hbm,
              idx_v, pos_v, rows_v, out_v, gsems, osems):
    wid = lax.axis_index("s") * 2 + lax.axis_index("c")
    pltpu.sync_copy(ids_hbm.at[pl.ds(wid * NCHUNK, NCHUNK)], idx_v)
    pltpu.sync_copy(pos_hbm, pos_v)

    iota16 = lax.iota(jnp.int32, 16)
    fidx = [[lax.shift_right_logical(iota16 + h * LANES, 3) * 1024
             + lax.bitwise_and(iota16 + h * LANES, 7) * CHUNK + u
             for u in range(8)] for h in range(2)]

    def chunk_lbt(cc):
        g = wid * NCHUNK + cc
        l = lax.div(g, 8 * NBT) * 8 + lax.rem(g, 8)
        bt = lax.rem(lax.div(g, 8), NBT)
        return l, bt

    def gather(cc, p):
        return pltpu.make_async_copy(
            table_hbm.at[idx_v.at[cc]], rows_v[p], gsems[p])

    def wb(cc, p):
        l, bt = chunk_lbt(cc)
        return [pltpu.make_async_copy(
                    out_v[p].at[pl.ds(dt * 1024, 1024)],
                    out_hbm.at[l, pl.ds((dt * NBT + bt) * 1024, 1024)],
                    osems[p])
                for dt in range(DT)]

    for k in range(NBUF - 1):
        gather(k, k).start()

    def step(i, carry):
        for b in range(NBUF):
            cc = i * NBUF + b

            @pl.when(cc >= NBUF)
            def _drain():
                for c in wb(cc - NBUF, b):
                    c.wait()

            @pl.when(cc + NBUF - 1 < NCHUNK)
            def _prefetch():
                gather(cc + NBUF - 1, (b + NBUF - 1) % NBUF).start()

            gather(cc, b).wait()

            l, _ = chunk_lbt(cc)
            pos_c = [pos_v[pl.ds(l * EMBED + h * LANES, LANES)]
                     for h in range(2)]

            @plsc.parallel_loop(0, CHUNK, step=8)
            def tok_body(j):
                bjj = jnp.full((LANES,), j, jnp.int32)
                vals = [rows_v[b][j + u, pl.ds(h * LANES, LANES)] + pos_c[h]
                        for u in range(8) for h in range(2)]
                for u in range(8):
                    for h in range(2):
                        plsc.store_scatter(out_v[b], [fidx[h][u] + bjj],
                                           vals[u * 2 + h])

            for c in wb(cc, b):
                c.start()
        return carry

    lax.fori_loop(0, NCHUNK // NBUF, step, 0)

    for k in range(NBUF):
        cc = NCHUNK - NBUF + k
        for c in wb(cc, cc % NBUF):
            c.wait()


@jax.jit
def _emb_call(ids, pos, table4):
    mesh = plsc.VectorSubcoreMesh(core_axis_name="c", subcore_axis_name="s")
    f = pl.kernel(
        _emb_body,
        out_type=jax.ShapeDtypeStruct((LENGTH, DT * NBT * 8 * CHUNK),
                                      jnp.float32),
        mesh=mesh,
        compiler_params=pltpu.CompilerParams(use_tc_tiling_on_sc=False,
                                             needs_layout_passes=False),
        scratch_types=[
            pltpu.VMEM((NCHUNK, CHUNK), jnp.int32),
            pltpu.VMEM((LENGTH * EMBED,), jnp.float32),
            [pltpu.VMEM((CHUNK, EMBED), jnp.float32) for _ in range(NBUF)],
            [pltpu.VMEM((DT * 8 * CHUNK,), jnp.float32) for _ in range(NBUF)],
            [pltpu.SemaphoreType.DMA for _ in range(NBUF)],
            [pltpu.SemaphoreType.DMA for _ in range(NBUF)],
        ],
    )
    return f(ids, pos, table4)


def kernel(input_ids, token_table, position_table):
    ids = ((input_ids.astype(jnp.int32) * 4).T
           .reshape(LENGTH // 8, 8, NBT, CHUNK)
           .transpose(0, 2, 1, 3)
           .reshape(TOKENS // CHUNK, CHUNK))
    pos = position_table.reshape(LENGTH * EMBED)
    table4 = jnp.pad(token_table, ((0, 0), (0, 96))).reshape(4 * VOCAB, EMBED)
    out2 = _emb_call(ids, pos, table4)
    out5 = out2.reshape(LENGTH, DT, NBT, 8, CHUNK)
    return out5.transpose(2, 4, 0, 1, 3).reshape(BATCH, LENGTH, EMBED)


# speedup vs baseline: 1.1639x; 1.0017x over previous
import jax
import jax.numpy as jnp
from jax import lax
from jax.experimental import pallas as pl
from jax.experimental.pallas import tpu as pltpu
from jax.experimental.pallas import tpu_sc as plsc

VOCAB = 1000000
LENGTH = 200
EMBED = 32
BATCH = 4096

NW = 32
CHUNK = 128
TOKENS = BATCH * LENGTH
PER_W = TOKENS // NW
NCHUNK = PER_W // CHUNK
LANES = 16
NBT = BATCH // CHUNK
DT = EMBED // 8
NBUF = 4


def _emb_body(ids_hbm, pos_hbm, table_hbm, out_hbm,
              idx_v, pos_v, rows_v, out_v, gsems, osems):
    wid = lax.axis_index("s") * 2 + lax.axis_index("c")
    pltpu.sync_copy(ids_hbm.at[pl.ds(wid * NCHUNK, NCHUNK)], idx_v)
    pltpu.sync_copy(pos_hbm, pos_v)

    iota16 = lax.iota(jnp.int32, 16)
    fidx = [[lax.shift_right_logical(iota16 + h * LANES, 3) * 1024
             + lax.bitwise_and(iota16 + h * LANES, 7) * CHUNK + u
             for u in range(8)] for h in range(2)]

    def chunk_lbt(cc):
        g = wid * NCHUNK + cc
        l = lax.div(g, 8 * NBT) * 8 + lax.rem(g, 8)
        bt = lax.rem(lax.div(g, 8), NBT)
        return l, bt

    def gather(cc, p):
        return pltpu.make_async_copy(
            table_hbm.at[idx_v.at[cc]], rows_v[p], gsems[p])

    def wb(cc, p):
        l, bt = chunk_lbt(cc)
        return [pltpu.make_async_copy(
                    out_v[p].at[pl.ds(dt * 1024, 1024)],
                    out_hbm.at[l, pl.ds((dt * NBT + bt) * 1024, 1024)],
                    osems[p])
                for dt in range(DT)]

    for k in range(NBUF - 1):
        gather(k, k).start()

    def step(i, carry):
        for b in range(NBUF):
            cc = i * NBUF + b

            @pl.when(cc >= NBUF)
            def _drain():
                pltpu.make_async_copy(
                    out_hbm.at[0, pl.ds(0, DT * 8 * CHUNK)],
                    out_v[b], osems[b]).wait()

            @pl.when(cc + NBUF - 1 < NCHUNK)
            def _prefetch():
                gather(cc + NBUF - 1, (b + NBUF - 1) % NBUF).start()

            gather(cc, b).wait()

            l, _ = chunk_lbt(cc)
            pos_c = [pos_v[pl.ds(l * EMBED + h * LANES, LANES)]
                     for h in range(2)]

            @plsc.parallel_loop(0, CHUNK, step=8)
            def tok_body(j):
                bjj = jnp.full((LANES,), j, jnp.int32)
                vals = [rows_v[b][j + u, pl.ds(h * LANES, LANES)] + pos_c[h]
                        for u in range(8) for h in range(2)]
                for u in range(8):
                    for h in range(2):
                        plsc.store_scatter(out_v[b], [fidx[h][u] + bjj],
                                           vals[u * 2 + h])

            for c in wb(cc, b):
                c.start()
        return carry

    lax.fori_loop(0, NCHUNK // NBUF, step, 0)

    for k in range(NBUF):
        cc = NCHUNK - NBUF + k
        pltpu.make_async_copy(
            out_hbm.at[0, pl.ds(0, DT * 8 * CHUNK)],
            out_v[cc % NBUF], osems[cc % NBUF]).wait()


@jax.jit
def _emb_call(ids, pos, table4):
    mesh = plsc.VectorSubcoreMesh(core_axis_name="c", subcore_axis_name="s")
    f = pl.kernel(
        _emb_body,
        out_type=jax.ShapeDtypeStruct((LENGTH, DT * NBT * 8 * CHUNK),
                                      jnp.float32),
        mesh=mesh,
        compiler_params=pltpu.CompilerParams(use_tc_tiling_on_sc=False,
                                             needs_layout_passes=False),
        scratch_types=[
            pltpu.VMEM((NCHUNK, CHUNK), jnp.int32),
            pltpu.VMEM((LENGTH * EMBED,), jnp.float32),
            [pltpu.VMEM((CHUNK, EMBED), jnp.float32) for _ in range(NBUF)],
            [pltpu.VMEM((DT * 8 * CHUNK,), jnp.float32) for _ in range(NBUF)],
            [pltpu.SemaphoreType.DMA for _ in range(NBUF)],
            [pltpu.SemaphoreType.DMA for _ in range(NBUF)],
        ],
    )
    return f(ids, pos, table4)


def kernel(input_ids, token_table, position_table):
    ids = ((input_ids.astype(jnp.int32) * 4).T
           .reshape(LENGTH // 8, 8, NBT, CHUNK)
           .transpose(0, 2, 1, 3)
           .reshape(TOKENS // CHUNK, CHUNK))
    pos = position_table.reshape(LENGTH * EMBED)
    table4 = jnp.pad(token_table, ((0, 0), (0, 96))).reshape(4 * VOCAB, EMBED)
    out2 = _emb_call(ids, pos, table4)
    out5 = out2.reshape(LENGTH, DT, NBT, 8, CHUNK)
    return out5.transpose(2, 4, 0, 1, 3).reshape(BATCH, LENGTH, EMBED)


# speedup vs baseline: 1.7742x; 1.5243x over previous
import jax
import jax.numpy as jnp
from jax import lax
from jax.experimental import pallas as pl
from jax.experimental.pallas import tpu as pltpu
from jax.experimental.pallas import tpu_sc as plsc

VOCAB = 1000000
LENGTH = 200
EMBED = 32
BATCH = 4096

NW = 32
CHUNK = 128
TOKENS = BATCH * LENGTH
PER_W = TOKENS // NW
NCHUNK = PER_W // CHUNK
LANES = 16
NBT = BATCH // CHUNK
DT = EMBED // 8
NBUF = 4


def _emb_body(ids_hbm, pos_hbm, table_hbm, out_hbm,
              idx_v, pos_v, rows_v, out_v, gsems, osems):
    wid = lax.axis_index("s") * 2 + lax.axis_index("c")
    pltpu.sync_copy(ids_hbm.at[pl.ds(wid * NCHUNK, NCHUNK)], idx_v)
    pltpu.sync_copy(pos_hbm, pos_v)

    iota16 = lax.iota(jnp.int32, 16)
    dbase = (lax.shift_right_logical(iota16, 3) * 1024
             + lax.bitwise_and(iota16, 7) * CHUNK)
    rks, dsts = [], []
    for k in range(LANES):
        rk = lax.bitwise_and(iota16 + k, 15)
        rks.append(rk)
        dsts.append(dbase + rk)
    dlane = [iota16 + h * LANES for h in range(2)]

    def chunk_lbt(cc):
        g = wid * NCHUNK + cc
        l = lax.div(g, 8 * NBT) * 8 + lax.rem(g, 8)
        bt = lax.rem(lax.div(g, 8), NBT)
        return l, bt

    def gather(cc, p):
        return pltpu.make_async_copy(
            table_hbm.at[idx_v.at[cc]], rows_v[p], gsems[p])

    def wb(cc, p):
        l, bt = chunk_lbt(cc)
        return [pltpu.make_async_copy(
                    out_v[p].at[pl.ds(dt * 1024, 1024)],
                    out_hbm.at[l, pl.ds((dt * NBT + bt) * 1024, 1024)],
                    osems[p])
                for dt in range(DT)]

    for k in range(NBUF - 1):
        gather(k, k).start()

    def step(i, carry):
        for b in range(NBUF):
            cc = i * NBUF + b

            @pl.when(cc >= NBUF)
            def _drain():
                pltpu.make_async_copy(
                    out_hbm.at[0, pl.ds(0, DT * 8 * CHUNK)],
                    out_v[b], osems[b]).wait()

            @pl.when(cc + NBUF - 1 < NCHUNK)
            def _prefetch():
                gather(cc + NBUF - 1, (b + NBUF - 1) % NBUF).start()

            gather(cc, b).wait()

            l, _ = chunk_lbt(cc)
            pos_c = [pos_v[pl.ds(l * EMBED + h * LANES, LANES)]
                     for h in range(2)]

            @plsc.parallel_loop(0, CHUNK, step=LANES)
            def tok_body(j0):
                vj0 = jnp.full((LANES,), j0, jnp.int32)
                for h in range(2):
                    vdb = jnp.full((LANES,), h * 2048 + j0, jnp.int32)
                    vals = [plsc.load_gather(rows_v[b],
                                             [rks[k] + vj0, dlane[h]])
                            + pos_c[h] for k in range(LANES)]
                    for k in range(LANES):
                        plsc.store_scatter(out_v[b], [dsts[k] + vdb], vals[k])

            for c in wb(cc, b):
                c.start()
        return carry

    lax.fori_loop(0, NCHUNK // NBUF, step, 0)

    for k in range(NBUF):
        cc = NCHUNK - NBUF + k
        pltpu.make_async_copy(
            out_hbm.at[0, pl.ds(0, DT * 8 * CHUNK)],
            out_v[cc % NBUF], osems[cc % NBUF]).wait()


@jax.jit
def _emb_call(ids, pos, table4):
    mesh = plsc.VectorSubcoreMesh(core_axis_name="c", subcore_axis_name="s")
    f = pl.kernel(
        _emb_body,
        out_type=jax.ShapeDtypeStruct((LENGTH, DT * NBT * 8 * CHUNK),
                                      jnp.float32),
        mesh=mesh,
        compiler_params=pltpu.CompilerParams(use_tc_tiling_on_sc=False,
                                             needs_layout_passes=False),
        scratch_types=[
            pltpu.VMEM((NCHUNK, CHUNK), jnp.int32),
            pltpu.VMEM((LENGTH * EMBED,), jnp.float32),
            [pltpu.VMEM((CHUNK, EMBED), jnp.float32) for _ in range(NBUF)],
            [pltpu.VMEM((DT * 8 * CHUNK,), jnp.float32) for _ in range(NBUF)],
            [pltpu.SemaphoreType.DMA for _ in range(NBUF)],
            [pltpu.SemaphoreType.DMA for _ in range(NBUF)],
        ],
    )
    return f(ids, pos, table4)


def kernel(input_ids, token_table, position_table):
    ids = ((input_ids.astype(jnp.int32) * 4).T
           .reshape(LENGTH // 8, 8, NBT, CHUNK)
           .transpose(0, 2, 1, 3)
           .reshape(TOKENS // CHUNK, CHUNK))
    pos = position_table.reshape(LENGTH * EMBED)
    table4 = jnp.pad(token_table, ((0, 0), (0, 96))).reshape(4 * VOCAB, EMBED)
    out2 = _emb_call(ids, pos, table4)
    out5 = out2.reshape(LENGTH, DT, NBT, 8, CHUNK)
    return out5.transpose(2, 4, 0, 1, 3).reshape(BATCH, LENGTH, EMBED)


# speedup vs baseline: 2.5079x; 1.4136x over previous
import jax
import jax.numpy as jnp
from jax import lax
from jax.experimental import pallas as pl
from jax.experimental.pallas import tpu as pltpu
from jax.experimental.pallas import tpu_sc as plsc

VOCAB = 1000000
LENGTH = 200
EMBED = 32
BATCH = 4096

NW = 32
CHUNK = 128
TOKENS = BATCH * LENGTH
PER_W = TOKENS // NW
NCHUNK = PER_W // CHUNK
LANES = 16
NBT = BATCH // CHUNK
DT = EMBED // 8
NBUF = 4
NVT = 7813
VPAD = NVT * CHUNK


def _tr_body(tv_hbm, out_hbm, vin, vout, isems, osems):
    wid = lax.axis_index("s") * 2 + lax.axis_index("c")
    iota16 = lax.iota(jnp.int32, 16)
    rks = [lax.bitwise_and(iota16 + k, 15) for k in range(LANES)]
    dlane = [iota16 + h * LANES for h in range(2)]

    def in_copies(vt, p):
        return [pltpu.make_async_copy(
                    tv_hbm.at[dt, vt], vin[p].at[pl.ds(dt * 8, 8)], isems[p])
                for dt in range(DT)]

    def wb(vt, p):
        return pltpu.make_async_copy(
            vout[p], out_hbm.at[pl.ds(vt * CHUNK, CHUNK)], osems[p])

    def transpose(vi, vo):
        @plsc.parallel_loop(0, CHUNK, step=LANES)
        def _tile(vb):
            vvb = jnp.full((LANES,), vb, jnp.int32)
            for h in range(2):
                vvs = [rks[k] + vvb for k in range(LANES)]
                vals = [plsc.load_gather(vi, [dlane[h], vvs[k]])
                        for k in range(LANES)]
                for k in range(LANES):
                    plsc.store_scatter(vo, [vvs[k], dlane[h]], vals[k])

    NTI = NVT // NW
    for c in in_copies(wid, 0):
        c.start()

    def step(i2, carry):
        for b in range(2):
            i = i2 * 2 + b
            vt = i * NW + wid

            @pl.when(i >= 2)
            def _drain():
                wb(vt - 2 * NW, b).wait()

            @pl.when(i + 1 < NTI)
            def _pref():
                for c in in_copies(vt + NW, 1 - b):
                    c.start()

            for c in in_copies(vt, b):
                c.wait()
            transpose(vin[b], vout[b])
            wb(vt, b).start()
        return carry

    lax.fori_loop(0, NTI // 2, step, 0)
    wb((NTI - 2) * NW + wid, 0).wait()
    wb((NTI - 1) * NW + wid, 1).wait()

    @pl.when(wid < NVT - NTI * NW)
    def _tail():
        vt = NTI * NW + wid
        for dt in range(DT):
            pltpu.sync_copy(tv_hbm.at[dt, vt], vin[0].at[pl.ds(dt * 8, 8)])
        transpose(vin[0], vout[0])
        pltpu.sync_copy(vout[0], out_hbm.at[pl.ds(vt * CHUNK, CHUNK)])


def _emb_body(ids_hbm, pos_hbm, table_hbm, out_hbm,
              idx_v, pos_v, rows_v, out_v, gsems, osems):
    wid = lax.axis_index("s") * 2 + lax.axis_index("c")
    pltpu.sync_copy(ids_hbm.at[pl.ds(wid * NCHUNK, NCHUNK)], idx_v)
    pltpu.sync_copy(pos_hbm, pos_v)

    iota16 = lax.iota(jnp.int32, 16)
    dbase = (lax.shift_right_logical(iota16, 3) * 1024
             + lax.bitwise_and(iota16, 7) * CHUNK)
    rks, dsts = [], []
    for k in range(LANES):
        rk = lax.bitwise_and(iota16 + k, 15)
        rks.append(rk)
        dsts.append(dbase + rk)
    dlane = [iota16 + h * LANES for h in range(2)]

    def chunk_lbt(cc):
        g = wid * NCHUNK + cc
        l = lax.div(g, 8 * NBT) * 8 + lax.rem(g, 8)
        bt = lax.rem(lax.div(g, 8), NBT)
        return l, bt

    def gather(cc, p):
        return pltpu.make_async_copy(
            table_hbm.at[idx_v.at[cc]], rows_v[p], gsems[p])

    def wb(cc, p):
        l, bt = chunk_lbt(cc)
        return [pltpu.make_async_copy(
                    out_v[p].at[pl.ds(dt * 1024, 1024)],
                    out_hbm.at[l, pl.ds((dt * NBT + bt) * 1024, 1024)],
                    osems[p])
                for dt in range(DT)]

    for k in range(NBUF - 1):
        gather(k, k).start()

    def step(i, carry):
        for b in range(NBUF):
            cc = i * NBUF + b

            @pl.when(cc >= NBUF)
            def _drain():
                pltpu.make_async_copy(
                    out_hbm.at[0, pl.ds(0, DT * 8 * CHUNK)],
                    out_v[b], osems[b]).wait()

            @pl.when(cc + NBUF - 1 < NCHUNK)
            def _prefetch():
                gather(cc + NBUF - 1, (b + NBUF - 1) % NBUF).start()

            gather(cc, b).wait()

            l, _ = chunk_lbt(cc)
            pos_c = [pos_v[pl.ds(l * EMBED + h * LANES, LANES)]
                     for h in range(2)]

            @plsc.parallel_loop(0, CHUNK, step=LANES)
            def tok_body(j0):
                vj0 = jnp.full((LANES,), j0, jnp.int32)
                for h in range(2):
                    vdb = jnp.full((LANES,), h * 2048 + j0, jnp.int32)
                    vals = [plsc.load_gather(rows_v[b],
                                             [rks[k] + vj0, dlane[h]])
                            + pos_c[h] for k in range(LANES)]
                    for k in range(LANES):
                        plsc.store_scatter(out_v[b], [dsts[k] + vdb], vals[k])

            for c in wb(cc, b):
                c.start()
        return carry

    lax.fori_loop(0, NCHUNK // NBUF, step, 0)

    for k in range(NBUF):
        cc = NCHUNK - NBUF + k
        pltpu.make_async_copy(
            out_hbm.at[0, pl.ds(0, DT * 8 * CHUNK)],
            out_v[cc % NBUF], osems[cc % NBUF]).wait()


def _tr_call(tview):
    mesh = plsc.VectorSubcoreMesh(core_axis_name="c", subcore_axis_name="s")
    f = pl.kernel(
        _tr_body,
        out_type=jax.ShapeDtypeStruct((VPAD, EMBED), jnp.float32),
        mesh=mesh,
        compiler_params=pltpu.CompilerParams(use_tc_tiling_on_sc=False,
                                             needs_layout_passes=False),
        scratch_types=[
            [pltpu.VMEM((EMBED, CHUNK), jnp.float32) for _ in range(2)],
            [pltpu.VMEM((CHUNK, EMBED), jnp.float32) for _ in range(2)],
            [pltpu.SemaphoreType.DMA for _ in range(2)],
            [pltpu.SemaphoreType.DMA for _ in range(2)],
        ],
    )
    return f(tview)


@jax.jit
def _emb_call(ids, pos, tview):
    table4 = _tr_call(tview)
    mesh = plsc.VectorSubcoreMesh(core_axis_name="c", subcore_axis_name="s")
    f = pl.kernel(
        _emb_body,
        out_type=jax.ShapeDtypeStruct((LENGTH, DT * NBT * 8 * CHUNK),
                                      jnp.float32),
        mesh=mesh,
        compiler_params=pltpu.CompilerParams(use_tc_tiling_on_sc=False,
                                             needs_layout_passes=False),
        scratch_types=[
            pltpu.VMEM((NCHUNK, CHUNK), jnp.int32),
            pltpu.VMEM((LENGTH * EMBED,), jnp.float32),
            [pltpu.VMEM((CHUNK, EMBED), jnp.float32) for _ in range(NBUF)],
            [pltpu.VMEM((DT * 8 * CHUNK,), jnp.float32) for _ in range(NBUF)],
            [pltpu.SemaphoreType.DMA for _ in range(NBUF)],
            [pltpu.SemaphoreType.DMA for _ in range(NBUF)],
        ],
    )
    return f(ids, pos, table4)


def kernel(input_ids, token_table, position_table):
    ids = (input_ids.astype(jnp.int32).T
           .reshape(LENGTH // 8, 8, NBT, CHUNK)
           .transpose(0, 2, 1, 3)
           .reshape(TOKENS // CHUNK, CHUNK))
    pos = position_table.reshape(LENGTH * EMBED)
    tview = (jnp.pad(token_table, ((0, VPAD - VOCAB), (0, 0))).T
             .reshape(DT, 8, NVT, CHUNK)
             .transpose(0, 2, 1, 3))
    out2 = _emb_call(ids, pos, tview)
    out5 = out2.reshape(LENGTH, DT, NBT, 8, CHUNK)
    return out5.transpose(2, 4, 0, 1, 3).reshape(BATCH, LENGTH, EMBED)
